# baseline (speedup 1.0000x reference)
import dataclasses

import jax
import jax.numpy as jnp
from jax.experimental import pallas as pl
from jax.experimental.pallas import tpu as pltpu
from jax.experimental.pallas import tpu_sc as plsc

_SC_PARAMS = pltpu.CompilerParams()
if "needs_layout_passes" in pltpu.CompilerParams.__dataclass_fields__:
    _SC_PARAMS = dataclasses.replace(_SC_PARAMS, needs_layout_passes=False)

T = 4
N = 116
HID = 16
BERT = 768
E = 2048
D_CAT = BERT + HID
NEG = -1e30
NP = 128
SEG = (N * NP) // 8


def _hist_body(hg_ref, out_ref, vbuf, ebuf, hist_e, hist_v, sem):
    wid = jax.lax.axis_index("c") * 16 + jax.lax.axis_index("s")
    t = wid // 8
    w = wid % 8
    lo = w * SEG

    pltpu.async_copy(hg_ref.at[2 * t], vbuf, sem).wait()
    pltpu.async_copy(hg_ref.at[2 * t + 1], ebuf, sem).wait()

    zeros16 = jnp.zeros((16,), jnp.float32)

    @pl.loop(0, SEG, step=16)
    def _(i):
        hist_e[pl.ds(i, 16)] = zeros16
        hist_v[pl.ds(i, 16)] = zeros16

    ones16 = jnp.full((16,), 1.0, jnp.float32)

    @pl.loop(0, E, step=16)
    def _(j):
        v16 = vbuf[pl.ds(j, 16)]
        e16 = ebuf[pl.ds(j, 16)]
        key_e = e16 * NP + v16
        key_v = v16 * NP + e16
        me = (key_e >= lo) & (key_e < lo + SEG)
        mv = (key_v >= lo) & (key_v < lo + SEG)
        idx_e = jnp.where(me, key_e - lo, 0)
        idx_v = jnp.where(mv, key_v - lo, 0)
        plsc.addupdate_scatter(hist_e, [idx_e], ones16, mask=me)
        plsc.addupdate_scatter(hist_v, [idx_v], ones16, mask=mv)

    pltpu.async_copy(hist_e, out_ref.at[0, wid], sem).wait()
    pltpu.async_copy(hist_v, out_ref.at[1, wid], sem).wait()


def _histograms(hg2):
    k = pl.kernel(
        _hist_body,
        out_type=jax.ShapeDtypeStruct((2, 32, SEG), jnp.float32),
        mesh=plsc.VectorSubcoreMesh(core_axis_name="c", subcore_axis_name="s"),
        scratch_types=[
            pltpu.VMEM((E,), jnp.int32),
            pltpu.VMEM((E,), jnp.int32),
            pltpu.VMEM((SEG,), jnp.float32),
            pltpu.VMEM((SEG,), jnp.float32),
            pltpu.SemaphoreType.DMA,
        ],
        compiler_params=_SC_PARAMS,
    )
    return k(hg2)


def _lstm_gates(z, c):
    i = jax.nn.sigmoid(z[:, 0:HID])
    f = jax.nn.sigmoid(z[:, HID:2 * HID])
    g = jnp.tanh(z[:, 2 * HID:3 * HID])
    o = jax.nn.sigmoid(z[:, 3 * HID:4 * HID])
    c = f * c + i * g
    return o * jnp.tanh(c), c


def _tca_body(ne_ref, pr_ref, wih1_ref, whh1_ref, b1_ref, wvc_ref,
              wecs_ref, wec_ref, bec_ref, wih2_ref,
              np_ref, sv_ref, wc1_ref, bc_ref, aewc_ref, sc2_ref):
    f32 = jnp.float32
    cdims = (((1,), (1,)), ((), ()))

    h = jnp.zeros((N, HID), f32)
    c = jnp.zeros((N, HID), f32)
    for t in range(T):
        z = (jax.lax.dot_general(pr_ref[t], wih1_ref[...], cdims,
                                 preferred_element_type=f32)
             + jax.lax.dot_general(h, whh1_ref[...], cdims,
                                   preferred_element_type=f32)
             + b1_ref[...])
        h, c = _lstm_gates(z, c)
        np_ref[t] = h
        sv_ref[t:t + 1, :] = jax.lax.dot_general(
            wvc_ref[...], h, (((0,), (1,)), ((), ())),
            preferred_element_type=f32)

    wc = jax.lax.dot_general(wec_ref[...], wih2_ref[...], cdims,
                             preferred_element_type=f32)
    wc1_ref[...] = wc[0:HID]
    bc_ref[...] = jax.lax.dot_general(bec_ref[...], wih2_ref[...], cdims,
                                      preferred_element_type=f32)
    wc2 = wc[HID:]
    wecs2 = wecs_ref[HID:]
    for t in range(T):
        ae = ne_ref[t]
        aewc_ref[t] = jnp.dot(ae, wc2, preferred_element_type=f32)
        sc2_ref[t:t + 1, :] = jax.lax.dot_general(
            wecs2, ae, (((0,), (1,)), ((), ())),
            preferred_element_type=f32)


def _tcb_body(ch_ref, np_ref, sv_ref, wc1_ref, bc_ref, aewc_ref, sc2_ref,
              wecs_ref, whh2_ref, b2_ref, wqin_ref, wout_ref, wfc_ref,
              bfc_ref, out_ref):
    f32 = jnp.float32
    cdims = (((1,), (1,)), ((), ()))

    zin = []
    nps = []
    for t in range(T):
        cev = ch_ref[0, t][:, 0:N]
        cve = ch_ref[1, t][:, 0:N]
        pe = np_ref[t]
        nps.append(pe)
        sv_row = sv_ref[t:t + 1, :]
        m1 = jnp.max(jnp.where(cev > 0, sv_row, NEG), axis=1, keepdims=True)
        m1 = jnp.where(m1 > 0.5 * NEG, m1, 0.0)
        a1 = jnp.where(cev > 0, cev * jnp.exp(sv_row - m1), 0.0)
        den1 = jnp.sum(a1, axis=1, keepdims=True)
        he = jnp.dot(a1, pe, preferred_element_type=f32) / (den1 + 1e-9)

        sc_row = (jax.lax.dot_general(wecs_ref[0:HID], he,
                                      (((0,), (1,)), ((), ())),
                                      preferred_element_type=f32)
                  + sc2_ref[t:t + 1, :])
        m2 = jnp.max(jnp.where(cve > 0, sc_row, NEG), axis=1, keepdims=True)
        m2 = jnp.where(m2 > 0.5 * NEG, m2, 0.0)
        a2 = jnp.where(cve > 0, cve * jnp.exp(sc_row - m2), 0.0)
        den2 = jnp.sum(a2, axis=1, keepdims=True)
        hcw = (jnp.dot(he, wc1_ref[...], preferred_element_type=f32)
               + aewc_ref[t])
        aggw = jnp.dot(a2, hcw, preferred_element_type=f32) / (den2 + 1e-9)
        zin.append(jnp.where(den2 > 0, aggw + bc_ref[...], 0.0))

    h2 = jnp.zeros((N, HID), f32)
    c2 = jnp.zeros((N, HID), f32)
    la = []
    for t in range(T):
        z = (zin[t]
             + jax.lax.dot_general(h2, whh2_ref[...], cdims,
                                   preferred_element_type=f32)
             + b2_ref[...])
        h2, c2 = _lstm_gates(z, c2)
        la.append(h2 + nps[t])

    q = la[T - 1]
    qp = jax.lax.dot_general(q, wqin_ref[...], cdims,
                             preferred_element_type=f32)
    scores = [jnp.sum(qp * la[t], axis=1, keepdims=True) for t in range(T)]
    m = scores[0]
    for t in range(1, T):
        m = jnp.maximum(m, scores[t])
    ws = [jnp.exp(scores[t] - m) for t in range(T)]
    den = ws[0]
    for t in range(1, T):
        den = den + ws[t]
    mix = ws[0] * la[0]
    for t in range(1, T):
        mix = mix + ws[t] * la[t]
    mix = mix / den
    comb = jnp.tanh(
        jax.lax.dot_general(mix, wout_ref[:, 0:HID], cdims,
                            preferred_element_type=f32)
        + jax.lax.dot_general(q, wout_ref[:, HID:2 * HID], cdims,
                              preferred_element_type=f32))
    out_ref[...] = (jax.lax.dot_general(comb, wfc_ref[...], cdims,
                                        preferred_element_type=f32)
                    + bfc_ref[...])


def kernel(hgs, node_embs, prices, Wih1, Whh1, b1, w_vc, w_ec_score, W_ec,
           b_ec, Wih2, Whh2, b2, W_qin, W_out, W_fc, b_fc):
    f32 = jnp.float32
    hg2 = hgs.astype(jnp.int32).reshape(2 * T, E)

    np4, sv4, wc1, bc, aewc, sc2 = pl.pallas_call(
        _tca_body,
        out_shape=(
            jax.ShapeDtypeStruct((T, N, HID), f32),
            jax.ShapeDtypeStruct((T, N), f32),
            jax.ShapeDtypeStruct((HID, 4 * HID), f32),
            jax.ShapeDtypeStruct((1, 4 * HID), f32),
            jax.ShapeDtypeStruct((T, N, 4 * HID), f32),
            jax.ShapeDtypeStruct((T, N), f32),
        ),
    )(node_embs, prices, Wih1, Whh1, b1.reshape(1, 4 * HID),
      w_vc.reshape(HID, 1), w_ec_score.reshape(D_CAT, 1), W_ec,
      b_ec.reshape(1, D_CAT), Wih2)

    ch = _histograms(hg2).reshape(2, T, N, NP)

    return pl.pallas_call(
        _tcb_body,
        out_shape=jax.ShapeDtypeStruct((N, 2), f32),
    )(ch, np4, sv4, wc1, bc, aewc, sc2, w_ec_score.reshape(D_CAT, 1),
      Whh2, b2.reshape(1, 4 * HID), W_qin, W_out, W_fc,
      b_fc.reshape(1, 2))


# speedup vs baseline: 1.0908x; 1.0908x over previous
import dataclasses

import jax
import jax.numpy as jnp
from jax.experimental import pallas as pl
from jax.experimental.pallas import tpu as pltpu
from jax.experimental.pallas import tpu_sc as plsc

T = 4
N = 116
HID = 16
BERT = 768
E = 2048
D_CAT = BERT + HID
NEG = -1e30
NP = 128
SEG = (N * NP) // 8

_SC_PARAMS = pltpu.CompilerParams()
if "needs_layout_passes" in pltpu.CompilerParams.__dataclass_fields__:
    _SC_PARAMS = dataclasses.replace(_SC_PARAMS, needs_layout_passes=False)


def _hist_body(hg_ref, out_ref, vbuf, ebuf, hist_e, hist_v, sem1, sem2):
    wid = jax.lax.axis_index("c") * 16 + jax.lax.axis_index("s")
    t = wid // 8
    w = wid % 8
    lo = w * SEG

    cp1 = pltpu.async_copy(hg_ref.at[t, 0], vbuf, sem1)
    cp2 = pltpu.async_copy(hg_ref.at[t, 1], ebuf, sem2)

    zeros16 = jnp.zeros((16,), jnp.float32)

    @pl.loop(0, SEG, step=16)
    def _(i):
        hist_e[pl.ds(i, 16)] = zeros16
        hist_v[pl.ds(i, 16)] = zeros16

    cp1.wait()
    cp2.wait()

    ones16 = jnp.full((16,), 1.0, jnp.float32)

    @pl.loop(0, E, step=16)
    def _(j):
        v16 = vbuf[pl.ds(j, 16)]
        e16 = ebuf[pl.ds(j, 16)]
        key_e = e16 * NP + v16
        key_v = v16 * NP + e16
        me = (key_e >= lo) & (key_e < lo + SEG)
        mv = (key_v >= lo) & (key_v < lo + SEG)
        idx_e = jnp.where(me, key_e - lo, 0)
        idx_v = jnp.where(mv, key_v - lo, 0)
        plsc.addupdate_scatter(hist_e, [idx_e], ones16, mask=me)
        plsc.addupdate_scatter(hist_v, [idx_v], ones16, mask=mv)

    cp3 = pltpu.async_copy(hist_e, out_ref.at[0, wid], sem1)
    cp4 = pltpu.async_copy(hist_v, out_ref.at[1, wid], sem2)
    cp3.wait()
    cp4.wait()


def _histograms(hgs):
    k = pl.kernel(
        _hist_body,
        out_type=jax.ShapeDtypeStruct((2, 32, SEG), jnp.float32),
        mesh=plsc.VectorSubcoreMesh(core_axis_name="c", subcore_axis_name="s"),
        scratch_types=[
            pltpu.VMEM((E,), jnp.int32),
            pltpu.VMEM((E,), jnp.int32),
            pltpu.VMEM((SEG,), jnp.float32),
            pltpu.VMEM((SEG,), jnp.float32),
            pltpu.SemaphoreType.DMA,
            pltpu.SemaphoreType.DMA,
        ],
        compiler_params=_SC_PARAMS,
    )
    return k(hgs)


def _lstm_gates(z, c):
    i = jax.nn.sigmoid(z[:, 0:HID])
    f = jax.nn.sigmoid(z[:, HID:2 * HID])
    g = jnp.tanh(z[:, 2 * HID:3 * HID])
    o = jax.nn.sigmoid(z[:, 3 * HID:4 * HID])
    c = f * c + i * g
    return o * jnp.tanh(c), c


def _tc_body(ch_ref, ne_ref, pr_ref, wih1_ref, whh1_ref, b1_ref, wvc_ref,
             wecs_ref, wec_ref, bec_ref, wih2_ref, whh2_ref, b2_ref,
             wqin_ref, wout_ref, wfc_ref, bfc_ref, out_ref):
    f32 = jnp.float32
    cdims = (((1,), (1,)), ((), ()))
    rowd = (((0,), (1,)), ((), ()))

    h = jnp.zeros((N, HID), f32)
    c = jnp.zeros((N, HID), f32)
    new_prices = []
    sv_rows = []
    for t in range(T):
        z = (jax.lax.dot_general(pr_ref[t], wih1_ref[...], cdims,
                                 preferred_element_type=f32)
             + jax.lax.dot_general(h, whh1_ref[...], cdims,
                                   preferred_element_type=f32)
             + b1_ref[...])
        h, c = _lstm_gates(z, c)
        new_prices.append(h)
        sv_rows.append(jax.lax.dot_general(wvc_ref[...], h, rowd,
                                           preferred_element_type=f32))

    wc = jax.lax.dot_general(wec_ref[...], wih2_ref[...], cdims,
                             preferred_element_type=f32)
    bc = jax.lax.dot_general(bec_ref[...], wih2_ref[...], cdims,
                             preferred_element_type=f32)

    zin = []
    for t in range(T):
        cev = ch_ref[0, t][:, 0:N]
        cve = ch_ref[1, t][:, 0:N]
        pe = new_prices[t]
        sv_row = sv_rows[t]
        m1 = jnp.max(jnp.where(cev > 0, sv_row, NEG), axis=1, keepdims=True)
        m1 = jnp.where(m1 > 0.5 * NEG, m1, 0.0)
        a1 = jnp.where(cev > 0, cev * jnp.exp(sv_row - m1), 0.0)
        den1 = jnp.sum(a1, axis=1, keepdims=True)
        he = jnp.dot(a1, pe, preferred_element_type=f32) / (den1 + 1e-9)

        ae = ne_ref[t]
        sc_row = (jax.lax.dot_general(wecs_ref[0:HID], he, rowd,
                                      preferred_element_type=f32)
                  + jax.lax.dot_general(wecs_ref[HID:], ae, rowd,
                                        preferred_element_type=f32))
        m2 = jnp.max(jnp.where(cve > 0, sc_row, NEG), axis=1, keepdims=True)
        m2 = jnp.where(m2 > 0.5 * NEG, m2, 0.0)
        a2 = jnp.where(cve > 0, cve * jnp.exp(sc_row - m2), 0.0)
        den2 = jnp.sum(a2, axis=1, keepdims=True)
        hcw = (jnp.dot(he, wc[0:HID], preferred_element_type=f32)
               + jnp.dot(ae, wc[HID:], preferred_element_type=f32))
        aggw = jnp.dot(a2, hcw, preferred_element_type=f32) / (den2 + 1e-9)
        zin.append(jnp.where(den2 > 0, aggw + bc, 0.0))

    h2 = jnp.zeros((N, HID), f32)
    c2 = jnp.zeros((N, HID), f32)
    la = []
    for t in range(T):
        z = (zin[t]
             + jax.lax.dot_general(h2, whh2_ref[...], cdims,
                                   preferred_element_type=f32)
             + b2_ref[...])
        h2, c2 = _lstm_gates(z, c2)
        la.append(h2 + new_prices[t])

    q = la[T - 1]
    qp = jax.lax.dot_general(q, wqin_ref[...], cdims,
                             preferred_element_type=f32)
    scores = [jnp.sum(qp * la[t], axis=1, keepdims=True) for t in range(T)]
    m = scores[0]
    for t in range(1, T):
        m = jnp.maximum(m, scores[t])
    ws = [jnp.exp(scores[t] - m) for t in range(T)]
    den = ws[0]
    for t in range(1, T):
        den = den + ws[t]
    mix = ws[0] * la[0]
    for t in range(1, T):
        mix = mix + ws[t] * la[t]
    mix = mix / den
    comb = jnp.tanh(
        jax.lax.dot_general(mix, wout_ref[:, 0:HID], cdims,
                            preferred_element_type=f32)
        + jax.lax.dot_general(q, wout_ref[:, HID:2 * HID], cdims,
                              preferred_element_type=f32))
    out_ref[...] = (jax.lax.dot_general(comb, wfc_ref[...], cdims,
                                        preferred_element_type=f32)
                    + bfc_ref[...])


def kernel(hgs, node_embs, prices, Wih1, Whh1, b1, w_vc, w_ec_score, W_ec,
           b_ec, Wih2, Whh2, b2, W_qin, W_out, W_fc, b_fc):
    f32 = jnp.float32
    ch = _histograms(hgs.astype(jnp.int32)).reshape(2, T, N, NP)

    return pl.pallas_call(
        _tc_body,
        out_shape=jax.ShapeDtypeStruct((N, 2), f32),
    )(ch, node_embs, prices, Wih1, Whh1, b1.reshape(1, 4 * HID),
      w_vc.reshape(HID, 1), w_ec_score.reshape(D_CAT, 1), W_ec,
      b_ec.reshape(1, D_CAT), Wih2, Whh2, b2.reshape(1, 4 * HID),
      W_qin, W_out, W_fc, b_fc.reshape(1, 2))
